# baseline (speedup 1.0000x reference)
import jax
import jax.numpy as jnp
from jax import lax
from jax.experimental import pallas as pl
from jax.experimental.pallas import tpu as pltpu
from jax.experimental.pallas import tpu_sc as plsc

_B, _N, _D, _DFF, _E = 1024, 5, 256, 512, 8
_TEMP = 0.07
_T = _B * _N
_VE = 2 * _E
_G = 256
_NPAD = 4 * _T + _VE * _G
_NB = _NPAD // _G
_CS = 512
_BB = 256
_NW = 32
_RPW = 4 * _T // _NW
_CH = _RPW // 2


def _mm(a, b):
    return lax.dot_general(a, b, (((1,), (0,)), ((), ())),
                           preferred_element_type=jnp.float32)


def _top2(logits):
    ii = lax.broadcasted_iota(jnp.int32, logits.shape, 1)
    m1 = jnp.max(logits, axis=1, keepdims=True)
    i1 = jnp.min(jnp.where(logits == m1, ii, _E), axis=1, keepdims=True)
    oh1 = ii == i1
    l2 = jnp.where(oh1, -jnp.inf, logits)
    m2 = jnp.max(l2, axis=1, keepdims=True)
    i2 = jnp.min(jnp.where(l2 == m2, ii, _E), axis=1, keepdims=True)
    oh2 = ii == i2
    e21 = jnp.exp(m2 - m1)
    g1 = 1.0 / (1.0 + e21)
    g2 = e21 * g1
    return oh1, oh2, g1, g2


def _route_one(x, Wr):
    logits = _mm(x, Wr)
    oh1, oh2, g1, g2 = _top2(logits)
    oh = oh1.astype(jnp.float32) + oh2.astype(jnp.float32)
    r = lax.broadcasted_iota(jnp.int32, (_CS, _CS), 0)
    c = lax.broadcasted_iota(jnp.int32, (_CS, _CS), 1)
    ltri = (c < r).astype(jnp.bfloat16)
    chunks = []
    carry = jnp.zeros((1, _E), jnp.float32)
    for k in range(_T // _CS):
        ohc = oh[k * _CS:(k + 1) * _CS]
        cum = _mm(ltri, ohc.astype(jnp.bfloat16)) + carry
        chunks.append(cum)
        carry = carry + jnp.sum(ohc, axis=0, keepdims=True)
    cum_excl = jnp.concatenate(chunks, axis=0)
    rank1 = jnp.sum(jnp.where(oh1, cum_excl, 0.0), axis=1, keepdims=True)
    rank2 = jnp.sum(jnp.where(oh2, cum_excl, 0.0), axis=1, keepdims=True)
    return oh1, oh2, g1, g2, rank1, rank2, carry


def _router_body(xt_ref, xl_ref, wrt_ref, wrl_ref,
                 dest_ref, gates_ref, be_ref):
    oh1t, oh2t, g1t, g2t, r1t, r2t, cnt_t = _route_one(xt_ref[...], wrt_ref[...])
    oh1l, oh2l, g1l, g2l, r1l, r2l, cnt_l = _route_one(xl_ref[...], wrl_ref[...])

    counts = jnp.concatenate([cnt_t, cnt_l], axis=1).astype(jnp.int32)
    padded = ((counts + (_G - 1)) // _G) * _G
    s = padded
    for sh in (1, 2, 4, 8):
        s = s + jnp.concatenate(
            [jnp.zeros((1, sh), jnp.int32), s[:, :_VE - sh]], axis=1)
    start = s - padded
    start_t, start_l = start[:, :_E], start[:, _E:]

    def dest(oh, rank, st):
        base = jnp.sum(jnp.where(oh, jnp.broadcast_to(st.astype(jnp.float32),
                                                      oh.shape), 0.0),
                       axis=1, keepdims=True)
        return (base + rank).astype(jnp.int32)

    dest_ref[0 * _T:1 * _T] = dest(oh1t, r1t, start_t)
    dest_ref[1 * _T:2 * _T] = dest(oh2t, r2t, start_t)
    dest_ref[2 * _T:3 * _T] = dest(oh1l, r1l, start_l)
    dest_ref[3 * _T:4 * _T] = dest(oh2l, r2l, start_l)
    gates_ref[0 * _T:1 * _T] = g1t
    gates_ref[1 * _T:2 * _T] = g2t
    gates_ref[2 * _T:3 * _T] = g1l
    gates_ref[3 * _T:4 * _T] = g2l

    blkstart = start // _G
    bb = lax.broadcasted_iota(jnp.int32, (1, _NB), 1)
    bexp = jnp.full((1, _NB), -1, jnp.int32)
    for e in range(_VE):
        bexp = bexp + (bb >= blkstart[:, e:e + 1]).astype(jnp.int32)
    be_ref[...] = bexp


def _router(xt, xl, wrt, wrl):
    full = lambda shape: pl.BlockSpec(shape, lambda: (0,) * len(shape))
    return pl.pallas_call(
        _router_body,
        in_specs=[full((_T, _D)), full((_T, _D)),
                  full((_D, _E)), full((_D, _E))],
        out_specs=[full((4 * _T, 1)), full((4 * _T, 1)), full((1, _NB))],
        out_shape=[jax.ShapeDtypeStruct((4 * _T, 1), jnp.int32),
                   jax.ShapeDtypeStruct((4 * _T, 1), jnp.float32),
                   jax.ShapeDtypeStruct((1, _NB), jnp.int32)],
    )(xt, xl, wrt, wrl)


def _sc_mesh():
    return plsc.VectorSubcoreMesh(core_axis_name="c", subcore_axis_name="s",
                                  num_cores=2, num_subcores=16)


def _sc_dispatch(xt, xl, dest_flat):
    def body(xt_hbm, xl_hbm, dest_hbm, out_hbm, idx_v, rows_v, sem):
        wid = lax.axis_index("s") * 2 + lax.axis_index("c")
        for ch in range(2):
            gbase = wid * _RPW + ch * _CH
            t0 = lax.rem(gbase, _T)
            pltpu.sync_copy(dest_hbm.at[pl.ds(gbase, _CH)], idx_v)

            @pl.when(gbase < 2 * _T)
            def _():
                pltpu.sync_copy(xt_hbm.at[pl.ds(t0, _CH)], rows_v)

            @pl.when(gbase >= 2 * _T)
            def _():
                pltpu.sync_copy(xl_hbm.at[pl.ds(t0, _CH)], rows_v)

            pltpu.async_copy(rows_v, out_hbm.at[idx_v], sem).wait()

    return pl.kernel(
        body,
        out_type=jax.ShapeDtypeStruct((_NPAD, _D), jnp.float32),
        mesh=_sc_mesh(),
        scratch_types=[pltpu.VMEM((_CH,), jnp.int32),
                       pltpu.VMEM((_CH, _D), jnp.float32),
                       pltpu.SemaphoreType.DMA],
    )(xt, xl, dest_flat)


def _sc_gather(ys, dest_flat):
    def body(ys_hbm, dest_hbm, out_hbm, idx_v, rows_v, sem):
        wid = lax.axis_index("s") * 2 + lax.axis_index("c")
        for ch in range(2):
            gbase = wid * _RPW + ch * _CH
            pltpu.sync_copy(dest_hbm.at[pl.ds(gbase, _CH)], idx_v)
            pltpu.async_copy(ys_hbm.at[idx_v], rows_v, sem).wait()
            pltpu.sync_copy(rows_v, out_hbm.at[pl.ds(gbase, _CH)])

    return pl.kernel(
        body,
        out_type=jax.ShapeDtypeStruct((4 * _T, _D), jnp.float32),
        mesh=_sc_mesh(),
        scratch_types=[pltpu.VMEM((_CH,), jnp.int32),
                       pltpu.VMEM((_CH, _D), jnp.float32),
                       pltpu.SemaphoreType.DMA],
    )(ys, dest_flat)


def _ffn_body(be_ref, xs_ref, w1_ref, w2_ref, out_ref):
    xb = xs_ref[...].astype(jnp.bfloat16)
    h = jax.nn.gelu(_mm(xb, w1_ref[0]).astype(jnp.bfloat16))
    out_ref[...] = _mm(h, w2_ref[0])


def _ffn(xs, w1cat, w2cat, bexp):
    spec = pltpu.PrefetchScalarGridSpec(
        num_scalar_prefetch=1,
        grid=(_NB,),
        in_specs=[
            pl.BlockSpec((_G, _D), lambda b, be: (b, 0)),
            pl.BlockSpec((1, _D, _DFF), lambda b, be: (be[0, b], 0, 0)),
            pl.BlockSpec((1, _DFF, _D), lambda b, be: (be[0, b], 0, 0)),
        ],
        out_specs=pl.BlockSpec((_G, _D), lambda b, be: (b, 0)),
    )
    return pl.pallas_call(
        _ffn_body,
        grid_spec=spec,
        out_shape=jax.ShapeDtypeStruct((_NPAD, _D), jnp.float32),
    )(bexp, xs, w1cat, w2cat)


def _l2n(x):
    n = jnp.sqrt(jnp.sum(x * x, axis=-1, keepdims=True))
    return x / jnp.maximum(n, 1e-12)


def _loss_body(q_ref, items_ref, yt1_ref, yt2_ref, yl1_ref, yl2_ref,
               gt1_ref, gt2_ref, gl1_ref, gl2_ref, out_ref):
    i = pl.program_id(0)

    @pl.when(i == 0)
    def _():
        out_ref[...] = jnp.zeros((1, 1), jnp.float32)

    yt = (gt1_ref[0] * yt1_ref[0] + gt2_ref[0] * yt2_ref[0]).reshape(_BB, _N, _D)
    yl = (gl1_ref[0] * yl1_ref[0] + gl2_ref[0] * yl2_ref[0]).reshape(_BB, _N, _D)
    items = items_ref[...]
    q = q_ref[...]
    pos = jnp.concatenate(
        [q[:, None, :], items[:, 0:1], yl[:, 0:1], yt[:, 0:1]], axis=1)
    neg = jnp.concatenate([items[:, 1:], yl[:, 1:], yt[:, 1:]], axis=1)
    pos = _l2n(pos)
    neg = _l2n(neg)
    allf = jnp.concatenate([pos, neg], axis=1)

    iota_a = lax.broadcasted_iota(jnp.int32, (_BB, 4), 1)
    exp_pos = jnp.zeros((_BB, 4), jnp.float32)
    exp_neg = jnp.zeros((_BB, 4), jnp.float32)
    for k in range(16):
        s = jnp.sum(pos * allf[:, k:k + 1, :], axis=2)
        ek = jnp.exp(s / _TEMP)
        if k < 4:
            exp_pos = exp_pos + jnp.where(iota_a == k, 0.0, ek)
        else:
            exp_neg = exp_neg + ek
    ratio = exp_pos / (exp_pos + exp_neg + 1e-8)
    ratio = jnp.where(jnp.isnan(ratio), 0.0, ratio)
    out_ref[...] += -jnp.sum(jnp.log(ratio)).reshape(1, 1)


def _loss(query_emb, items_emb, yg4, gates4):
    tb = _BB * _N
    yspec = lambda row: pl.BlockSpec((1, tb, _D), lambda i, r=row: (r, i, 0))
    gspec = lambda row: pl.BlockSpec((1, tb, 1), lambda i, r=row: (r, i, 0))
    out = pl.pallas_call(
        _loss_body,
        grid=(_B // _BB,),
        in_specs=[
            pl.BlockSpec((_BB, _D), lambda i: (i, 0)),
            pl.BlockSpec((_BB, _N, _D), lambda i: (i, 0, 0)),
            yspec(0), yspec(1), yspec(2), yspec(3),
            gspec(0), gspec(1), gspec(2), gspec(3),
        ],
        out_specs=pl.BlockSpec((1, 1), lambda i: (0, 0)),
        out_shape=jax.ShapeDtypeStruct((1, 1), jnp.float32),
    )(query_emb, items_emb, yg4, yg4, yg4, yg4,
      gates4, gates4, gates4, gates4)
    return out[0, 0] / (_B * 4)


def kernel(query_emb, items_emb, item_text_tokens, item_llm_tokens,
           Wr_text, W1_text, W2_text, Wr_llm, W1_llm, W2_llm):
    xt = item_text_tokens.reshape(_T, _D)
    xl = item_llm_tokens.reshape(_T, _D)
    dest, gates, bexp = _router(xt, xl, Wr_text, Wr_llm)
    dest_flat = dest.reshape(4 * _T)
    xs = _sc_dispatch(xt, xl, dest_flat)
    w1cat = jnp.concatenate([W1_text, W1_llm], axis=0).astype(jnp.bfloat16)
    w2cat = jnp.concatenate([W2_text, W2_llm], axis=0).astype(jnp.bfloat16)
    ys = _ffn(xs, w1cat, w2cat, bexp)
    yg = _sc_gather(ys, dest_flat)
    yg4 = yg.reshape(4, _T, _D)
    gates4 = gates.reshape(4, _T, 1)
    return _loss(query_emb, items_emb, yg4, gates4)


# speedup vs baseline: 1.0019x; 1.0019x over previous
import jax
import jax.numpy as jnp
from jax import lax
from jax.experimental import pallas as pl
from jax.experimental.pallas import tpu as pltpu
from jax.experimental.pallas import tpu_sc as plsc

_B, _N, _D, _DFF, _E = 1024, 5, 256, 512, 8
_TEMP = 0.07
_T = _B * _N
_VE = 2 * _E
_G = 256
_NPAD = 4 * _T + _VE * _G
_NB = _NPAD // _G
_CS = 512
_BB = 256
_NW = 32
_RPW = 4 * _T // _NW
_CH = _RPW // 2


def _mm(a, b):
    return lax.dot_general(a, b, (((1,), (0,)), ((), ())),
                           preferred_element_type=jnp.float32)


def _top2(logits):
    ii = lax.broadcasted_iota(jnp.int32, logits.shape, 1)
    m1 = jnp.max(logits, axis=1, keepdims=True)
    i1 = jnp.min(jnp.where(logits == m1, ii, _E), axis=1, keepdims=True)
    oh1 = ii == i1
    l2 = jnp.where(oh1, -jnp.inf, logits)
    m2 = jnp.max(l2, axis=1, keepdims=True)
    i2 = jnp.min(jnp.where(l2 == m2, ii, _E), axis=1, keepdims=True)
    oh2 = ii == i2
    e21 = jnp.exp(m2 - m1)
    g1 = 1.0 / (1.0 + e21)
    g2 = e21 * g1
    return oh1, oh2, g1, g2


def _route_one(x, Wr):
    logits = _mm(x, Wr)
    oh1, oh2, g1, g2 = _top2(logits)
    oh = oh1.astype(jnp.float32) + oh2.astype(jnp.float32)
    r = lax.broadcasted_iota(jnp.int32, (_CS, _CS), 0)
    c = lax.broadcasted_iota(jnp.int32, (_CS, _CS), 1)
    ltri = (c < r).astype(jnp.bfloat16)
    chunks = []
    carry = jnp.zeros((1, _E), jnp.float32)
    for k in range(_T // _CS):
        ohc = oh[k * _CS:(k + 1) * _CS]
        cum = _mm(ltri, ohc.astype(jnp.bfloat16)) + carry
        chunks.append(cum)
        carry = carry + jnp.sum(ohc, axis=0, keepdims=True)
    cum_excl = jnp.concatenate(chunks, axis=0)
    rank1 = jnp.sum(jnp.where(oh1, cum_excl, 0.0), axis=1, keepdims=True)
    rank2 = jnp.sum(jnp.where(oh2, cum_excl, 0.0), axis=1, keepdims=True)
    return oh1, oh2, g1, g2, rank1, rank2, carry


def _router_body(xt_ref, xl_ref, wrt_ref, wrl_ref,
                 dest_ref, gates_ref, be_ref):
    oh1t, oh2t, g1t, g2t, r1t, r2t, cnt_t = _route_one(xt_ref[...], wrt_ref[...])
    oh1l, oh2l, g1l, g2l, r1l, r2l, cnt_l = _route_one(xl_ref[...], wrl_ref[...])

    counts = jnp.concatenate([cnt_t, cnt_l], axis=1).astype(jnp.int32)
    padded = ((counts + (_G - 1)) // _G) * _G
    s = padded
    for sh in (1, 2, 4, 8):
        s = s + jnp.concatenate(
            [jnp.zeros((1, sh), jnp.int32), s[:, :_VE - sh]], axis=1)
    start = s - padded
    start_t, start_l = start[:, :_E], start[:, _E:]

    def dest(oh, rank, st):
        base = jnp.sum(jnp.where(oh, jnp.broadcast_to(st.astype(jnp.float32),
                                                      oh.shape), 0.0),
                       axis=1, keepdims=True)
        return (base + rank).astype(jnp.int32)

    dest_ref[0 * _T:1 * _T] = dest(oh1t, r1t, start_t)
    dest_ref[1 * _T:2 * _T] = dest(oh2t, r2t, start_t)
    dest_ref[2 * _T:3 * _T] = dest(oh1l, r1l, start_l)
    dest_ref[3 * _T:4 * _T] = dest(oh2l, r2l, start_l)
    gates_ref[0 * _T:1 * _T] = g1t
    gates_ref[1 * _T:2 * _T] = g2t
    gates_ref[2 * _T:3 * _T] = g1l
    gates_ref[3 * _T:4 * _T] = g2l

    blkstart = start // _G
    bb = lax.broadcasted_iota(jnp.int32, (1, _NB), 1)
    bexp = jnp.full((1, _NB), -1, jnp.int32)
    for e in range(_VE):
        bexp = bexp + (bb >= blkstart[:, e:e + 1]).astype(jnp.int32)
    be_ref[...] = bexp


def _router(xt, xl, wrt, wrl):
    full = lambda shape: pl.BlockSpec(shape, lambda: (0,) * len(shape))
    return pl.pallas_call(
        _router_body,
        in_specs=[full((_T, _D)), full((_T, _D)),
                  full((_D, _E)), full((_D, _E))],
        out_specs=[full((4 * _T, 1)), full((4 * _T, 1)), full((1, _NB))],
        out_shape=[jax.ShapeDtypeStruct((4 * _T, 1), jnp.int32),
                   jax.ShapeDtypeStruct((4 * _T, 1), jnp.float32),
                   jax.ShapeDtypeStruct((1, _NB), jnp.int32)],
    )(xt, xl, wrt, wrl)


def _sc_mesh():
    return plsc.VectorSubcoreMesh(core_axis_name="c", subcore_axis_name="s",
                                  num_cores=2, num_subcores=16)


def _sc_dispatch(xt, xl, dest_flat):
    def body(xt_hbm, xl_hbm, dest_hbm, out_hbm, idx_v, rows_v, sem):
        wid = lax.axis_index("s") * 2 + lax.axis_index("c")
        for ch in range(2):
            gbase = wid * _RPW + ch * _CH
            t0 = lax.rem(gbase, _T)
            pltpu.sync_copy(dest_hbm.at[pl.ds(gbase, _CH)], idx_v)

            @pl.when(gbase < 2 * _T)
            def _():
                pltpu.sync_copy(xt_hbm.at[pl.ds(t0, _CH)], rows_v)

            @pl.when(gbase >= 2 * _T)
            def _():
                pltpu.sync_copy(xl_hbm.at[pl.ds(t0, _CH)], rows_v)

            pltpu.async_copy(rows_v, out_hbm.at[idx_v], sem).wait()

    return pl.kernel(
        body,
        out_type=jax.ShapeDtypeStruct((_NPAD, _D), jnp.float32),
        mesh=_sc_mesh(),
        scratch_types=[pltpu.VMEM((_CH,), jnp.int32),
                       pltpu.VMEM((_CH, _D), jnp.float32),
                       pltpu.SemaphoreType.DMA],
    )(xt, xl, dest_flat)


def _sc_gather(ys, dest_flat):
    def body(ys_hbm, dest_hbm, out_hbm, idx_v, rows_v, sem):
        wid = lax.axis_index("s") * 2 + lax.axis_index("c")
        for ch in range(2):
            gbase = wid * _RPW + ch * _CH
            pltpu.sync_copy(dest_hbm.at[pl.ds(gbase, _CH)], idx_v)
            pltpu.async_copy(ys_hbm.at[idx_v], rows_v, sem).wait()
            pltpu.sync_copy(rows_v, out_hbm.at[pl.ds(gbase, _CH)])

    return pl.kernel(
        body,
        out_type=jax.ShapeDtypeStruct((4 * _T, _D), jnp.float32),
        mesh=_sc_mesh(),
        scratch_types=[pltpu.VMEM((_CH,), jnp.int32),
                       pltpu.VMEM((_CH, _D), jnp.float32),
                       pltpu.SemaphoreType.DMA],
    )(ys, dest_flat)


def _ffn_body(be_ref, xs_ref, w1_ref, w2_ref, out_ref):
    e = be_ref[0, pl.program_id(0)]
    xb = xs_ref[...].astype(jnp.bfloat16)
    h = jax.nn.gelu(_mm(xb, w1_ref[e]).astype(jnp.bfloat16))
    out_ref[...] = _mm(h, w2_ref[e])


def _ffn(xs, w1cat, w2cat, bexp):
    spec = pltpu.PrefetchScalarGridSpec(
        num_scalar_prefetch=1,
        grid=(_NB,),
        in_specs=[
            pl.BlockSpec((_G, _D), lambda b, be: (b, 0)),
            pl.BlockSpec((_VE, _D, _DFF), lambda b, be: (0, 0, 0)),
            pl.BlockSpec((_VE, _DFF, _D), lambda b, be: (0, 0, 0)),
        ],
        out_specs=pl.BlockSpec((_G, _D), lambda b, be: (b, 0)),
    )
    return pl.pallas_call(
        _ffn_body,
        grid_spec=spec,
        out_shape=jax.ShapeDtypeStruct((_NPAD, _D), jnp.float32),
    )(bexp, xs, w1cat, w2cat)


def _l2n(x):
    n = jnp.sqrt(jnp.sum(x * x, axis=-1, keepdims=True))
    return x / jnp.maximum(n, 1e-12)


def _loss_body(q_ref, items_ref, yt1_ref, yt2_ref, yl1_ref, yl2_ref,
               gt1_ref, gt2_ref, gl1_ref, gl2_ref, out_ref):
    i = pl.program_id(0)

    @pl.when(i == 0)
    def _():
        out_ref[...] = jnp.zeros((1, 1), jnp.float32)

    yt = (gt1_ref[0] * yt1_ref[0] + gt2_ref[0] * yt2_ref[0]).reshape(_BB, _N, _D)
    yl = (gl1_ref[0] * yl1_ref[0] + gl2_ref[0] * yl2_ref[0]).reshape(_BB, _N, _D)
    items = items_ref[...]
    q = q_ref[...]
    pos = jnp.concatenate(
        [q[:, None, :], items[:, 0:1], yl[:, 0:1], yt[:, 0:1]], axis=1)
    neg = jnp.concatenate([items[:, 1:], yl[:, 1:], yt[:, 1:]], axis=1)
    pos = _l2n(pos)
    neg = _l2n(neg)
    allf = jnp.concatenate([pos, neg], axis=1)

    iota_a = lax.broadcasted_iota(jnp.int32, (_BB, 4), 1)
    exp_pos = jnp.zeros((_BB, 4), jnp.float32)
    exp_neg = jnp.zeros((_BB, 4), jnp.float32)
    for k in range(16):
        s = jnp.sum(pos * allf[:, k:k + 1, :], axis=2)
        ek = jnp.exp(s / _TEMP)
        if k < 4:
            exp_pos = exp_pos + jnp.where(iota_a == k, 0.0, ek)
        else:
            exp_neg = exp_neg + ek
    ratio = exp_pos / (exp_pos + exp_neg + 1e-8)
    ratio = jnp.where(jnp.isnan(ratio), 0.0, ratio)
    out_ref[...] += -jnp.sum(jnp.log(ratio)).reshape(1, 1)


def _loss(query_emb, items_emb, yg4, gates4):
    tb = _BB * _N
    yspec = lambda row: pl.BlockSpec((1, tb, _D), lambda i, r=row: (r, i, 0))
    gspec = lambda row: pl.BlockSpec((1, tb, 1), lambda i, r=row: (r, i, 0))
    out = pl.pallas_call(
        _loss_body,
        grid=(_B // _BB,),
        in_specs=[
            pl.BlockSpec((_BB, _D), lambda i: (i, 0)),
            pl.BlockSpec((_BB, _N, _D), lambda i: (i, 0, 0)),
            yspec(0), yspec(1), yspec(2), yspec(3),
            gspec(0), gspec(1), gspec(2), gspec(3),
        ],
        out_specs=pl.BlockSpec((1, 1), lambda i: (0, 0)),
        out_shape=jax.ShapeDtypeStruct((1, 1), jnp.float32),
    )(query_emb, items_emb, yg4, yg4, yg4, yg4,
      gates4, gates4, gates4, gates4)
    return out[0, 0] / (_B * 4)


def kernel(query_emb, items_emb, item_text_tokens, item_llm_tokens,
           Wr_text, W1_text, W2_text, Wr_llm, W1_llm, W2_llm):
    xt = item_text_tokens.reshape(_T, _D)
    xl = item_llm_tokens.reshape(_T, _D)
    dest, gates, bexp = _router(xt, xl, Wr_text, Wr_llm)
    dest_flat = dest.reshape(4 * _T)
    xs = _sc_dispatch(xt, xl, dest_flat)
    w1cat = jnp.concatenate([W1_text, W1_llm], axis=0).astype(jnp.bfloat16)
    w2cat = jnp.concatenate([W2_text, W2_llm], axis=0).astype(jnp.bfloat16)
    ys = _ffn(xs, w1cat, w2cat, bexp)
    yg = _sc_gather(ys, dest_flat)
    yg4 = yg.reshape(4, _T, _D)
    gates4 = gates.reshape(4, _T, 1)
    return _loss(query_emb, items_emb, yg4, gates4)


# speedup vs baseline: 1.0913x; 1.0893x over previous
import jax
import jax.numpy as jnp
from jax import lax
from jax.experimental import pallas as pl
from jax.experimental.pallas import tpu as pltpu
from jax.experimental.pallas import tpu_sc as plsc

_B, _N, _D, _DFF, _E = 1024, 5, 256, 512, 8
_TEMP = 0.07
_T = _B * _N
_VE = 2 * _E
_G = 512
_NPAD = 4 * _T + _VE * _G
_NB = _NPAD // _G
_CS = 512
_BB = 256
_NW = 32
_RPW = 4 * _T // _NW
_CH = _RPW // 2


def _mm(a, b):
    return lax.dot_general(a, b, (((1,), (0,)), ((), ())),
                           preferred_element_type=jnp.float32)


def _top2(logits):
    ii = lax.broadcasted_iota(jnp.int32, logits.shape, 1)
    m1 = jnp.max(logits, axis=1, keepdims=True)
    i1 = jnp.min(jnp.where(logits == m1, ii, _E), axis=1, keepdims=True)
    oh1 = ii == i1
    l2 = jnp.where(oh1, -jnp.inf, logits)
    m2 = jnp.max(l2, axis=1, keepdims=True)
    i2 = jnp.min(jnp.where(l2 == m2, ii, _E), axis=1, keepdims=True)
    oh2 = ii == i2
    e21 = jnp.exp(m2 - m1)
    g1 = 1.0 / (1.0 + e21)
    g2 = e21 * g1
    return oh1, oh2, g1, g2


def _route_one(x, Wr):
    logits = _mm(x, Wr)
    oh1, oh2, g1, g2 = _top2(logits)
    oh = oh1.astype(jnp.float32) + oh2.astype(jnp.float32)
    r = lax.broadcasted_iota(jnp.int32, (_CS, _CS), 0)
    c = lax.broadcasted_iota(jnp.int32, (_CS, _CS), 1)
    ltri = (c < r).astype(jnp.bfloat16)
    chunks = []
    carry = jnp.zeros((1, _E), jnp.float32)
    for k in range(_T // _CS):
        ohc = oh[k * _CS:(k + 1) * _CS]
        cum = _mm(ltri, ohc.astype(jnp.bfloat16)) + carry
        chunks.append(cum)
        carry = carry + jnp.sum(ohc, axis=0, keepdims=True)
    cum_excl = jnp.concatenate(chunks, axis=0)
    rank1 = jnp.sum(jnp.where(oh1, cum_excl, 0.0), axis=1, keepdims=True)
    rank2 = jnp.sum(jnp.where(oh2, cum_excl, 0.0), axis=1, keepdims=True)
    return oh1, oh2, g1, g2, rank1, rank2, carry


def _router_body(xt_ref, xl_ref, wrt_ref, wrl_ref,
                 dest_ref, gates_ref, be_ref):
    oh1t, oh2t, g1t, g2t, r1t, r2t, cnt_t = _route_one(xt_ref[...], wrt_ref[...])
    oh1l, oh2l, g1l, g2l, r1l, r2l, cnt_l = _route_one(xl_ref[...], wrl_ref[...])

    counts = jnp.concatenate([cnt_t, cnt_l], axis=1).astype(jnp.int32)
    padded = ((counts + (_G - 1)) // _G) * _G
    s = padded
    for sh in (1, 2, 4, 8):
        s = s + jnp.concatenate(
            [jnp.zeros((1, sh), jnp.int32), s[:, :_VE - sh]], axis=1)
    start = s - padded
    start_t, start_l = start[:, :_E], start[:, _E:]

    def dest(oh, rank, st):
        base = jnp.sum(jnp.where(oh, jnp.broadcast_to(st.astype(jnp.float32),
                                                      oh.shape), 0.0),
                       axis=1, keepdims=True)
        return (base + rank).astype(jnp.int32)

    dest_ref[0 * _T:1 * _T] = dest(oh1t, r1t, start_t)
    dest_ref[1 * _T:2 * _T] = dest(oh2t, r2t, start_t)
    dest_ref[2 * _T:3 * _T] = dest(oh1l, r1l, start_l)
    dest_ref[3 * _T:4 * _T] = dest(oh2l, r2l, start_l)
    gates_ref[0 * _T:1 * _T] = g1t
    gates_ref[1 * _T:2 * _T] = g2t
    gates_ref[2 * _T:3 * _T] = g1l
    gates_ref[3 * _T:4 * _T] = g2l

    blkstart = start // _G
    bb = lax.broadcasted_iota(jnp.int32, (1, _NB), 1)
    bexp = jnp.full((1, _NB), -1, jnp.int32)
    for e in range(_VE):
        bexp = bexp + (bb >= blkstart[:, e:e + 1]).astype(jnp.int32)
    be_ref[...] = bexp


def _router(xt, xl, wrt, wrl):
    full = lambda shape: pl.BlockSpec(shape, lambda: (0,) * len(shape))
    return pl.pallas_call(
        _router_body,
        in_specs=[full((_T, _D)), full((_T, _D)),
                  full((_D, _E)), full((_D, _E))],
        out_specs=[full((4 * _T, 1)), full((4 * _T, 1)), full((1, _NB))],
        out_shape=[jax.ShapeDtypeStruct((4 * _T, 1), jnp.int32),
                   jax.ShapeDtypeStruct((4 * _T, 1), jnp.float32),
                   jax.ShapeDtypeStruct((1, _NB), jnp.int32)],
    )(xt, xl, wrt, wrl)


def _sc_mesh():
    return plsc.VectorSubcoreMesh(core_axis_name="c", subcore_axis_name="s",
                                  num_cores=2, num_subcores=16)


def _sc_dispatch(xt, xl, dest_flat):
    def body(xt_hbm, xl_hbm, dest_hbm, out_hbm, idx_v, rows_v, sem):
        wid = lax.axis_index("s") * 2 + lax.axis_index("c")
        for ch in range(2):
            gbase = wid * _RPW + ch * _CH
            t0 = lax.rem(gbase, _T)
            pltpu.sync_copy(dest_hbm.at[pl.ds(gbase, _CH)], idx_v)

            @pl.when(gbase < 2 * _T)
            def _():
                pltpu.sync_copy(xt_hbm.at[pl.ds(t0, _CH)], rows_v)

            @pl.when(gbase >= 2 * _T)
            def _():
                pltpu.sync_copy(xl_hbm.at[pl.ds(t0, _CH)], rows_v)

            pltpu.async_copy(rows_v, out_hbm.at[idx_v], sem).wait()

    return pl.kernel(
        body,
        out_type=jax.ShapeDtypeStruct((_NPAD, _D), jnp.float32),
        mesh=_sc_mesh(),
        scratch_types=[pltpu.VMEM((_CH,), jnp.int32),
                       pltpu.VMEM((_CH, _D), jnp.float32),
                       pltpu.SemaphoreType.DMA],
    )(xt, xl, dest_flat)


def _sc_gather(ys, dest_flat):
    def body(ys_hbm, dest_hbm, out_hbm, idx_v, rows_v, sem):
        wid = lax.axis_index("s") * 2 + lax.axis_index("c")
        for ch in range(2):
            gbase = wid * _RPW + ch * _CH
            pltpu.sync_copy(dest_hbm.at[pl.ds(gbase, _CH)], idx_v)
            pltpu.async_copy(ys_hbm.at[idx_v], rows_v, sem).wait()
            pltpu.sync_copy(rows_v, out_hbm.at[pl.ds(gbase, _CH)])

    return pl.kernel(
        body,
        out_type=jax.ShapeDtypeStruct((4 * _T, _D), jnp.float32),
        mesh=_sc_mesh(),
        scratch_types=[pltpu.VMEM((_CH,), jnp.int32),
                       pltpu.VMEM((_CH, _D), jnp.float32),
                       pltpu.SemaphoreType.DMA],
    )(ys, dest_flat)


def _ffn_body(be_ref, xs_ref, w1_ref, w2_ref, out_ref):
    e = be_ref[0, pl.program_id(0)]
    xb = xs_ref[...].astype(jnp.bfloat16)
    h = jax.nn.gelu(_mm(xb, w1_ref[e]).astype(jnp.bfloat16))
    out_ref[...] = _mm(h, w2_ref[e])


def _ffn(xs, w1cat, w2cat, bexp):
    spec = pltpu.PrefetchScalarGridSpec(
        num_scalar_prefetch=1,
        grid=(_NB,),
        in_specs=[
            pl.BlockSpec((_G, _D), lambda b, be: (b, 0)),
            pl.BlockSpec((_VE, _D, _DFF), lambda b, be: (0, 0, 0)),
            pl.BlockSpec((_VE, _DFF, _D), lambda b, be: (0, 0, 0)),
        ],
        out_specs=pl.BlockSpec((_G, _D), lambda b, be: (b, 0)),
    )
    return pl.pallas_call(
        _ffn_body,
        grid_spec=spec,
        out_shape=jax.ShapeDtypeStruct((_NPAD, _D), jnp.float32),
    )(bexp, xs, w1cat, w2cat)


def _l2n(x):
    n = jnp.sqrt(jnp.sum(x * x, axis=-1, keepdims=True))
    return x / jnp.maximum(n, 1e-12)


def _loss_body(q_ref, items_ref, yt1_ref, yt2_ref, yl1_ref, yl2_ref,
               gt1_ref, gt2_ref, gl1_ref, gl2_ref, out_ref):
    i = pl.program_id(0)

    @pl.when(i == 0)
    def _():
        out_ref[...] = jnp.zeros((1, 1), jnp.float32)

    yt = (gt1_ref[0] * yt1_ref[0] + gt2_ref[0] * yt2_ref[0]).reshape(_BB, _N, _D)
    yl = (gl1_ref[0] * yl1_ref[0] + gl2_ref[0] * yl2_ref[0]).reshape(_BB, _N, _D)
    items = items_ref[...]
    q = q_ref[...]
    pos = jnp.concatenate(
        [q[:, None, :], items[:, 0:1], yl[:, 0:1], yt[:, 0:1]], axis=1)
    neg = jnp.concatenate([items[:, 1:], yl[:, 1:], yt[:, 1:]], axis=1)
    pos = _l2n(pos)
    neg = _l2n(neg)
    allf = jnp.concatenate([pos, neg], axis=1)

    iota_a = lax.broadcasted_iota(jnp.int32, (_BB, 4), 1)
    exp_pos = jnp.zeros((_BB, 4), jnp.float32)
    exp_neg = jnp.zeros((_BB, 4), jnp.float32)
    for k in range(16):
        s = jnp.sum(pos * allf[:, k:k + 1, :], axis=2)
        ek = jnp.exp(s / _TEMP)
        if k < 4:
            exp_pos = exp_pos + jnp.where(iota_a == k, 0.0, ek)
        else:
            exp_neg = exp_neg + ek
    ratio = exp_pos / (exp_pos + exp_neg + 1e-8)
    ratio = jnp.where(jnp.isnan(ratio), 0.0, ratio)
    out_ref[...] += -jnp.sum(jnp.log(ratio)).reshape(1, 1)


def _loss(query_emb, items_emb, yg4, gates4):
    tb = _BB * _N
    yspec = lambda row: pl.BlockSpec((1, tb, _D), lambda i, r=row: (r, i, 0))
    gspec = lambda row: pl.BlockSpec((1, tb, 1), lambda i, r=row: (r, i, 0))
    out = pl.pallas_call(
        _loss_body,
        grid=(_B // _BB,),
        in_specs=[
            pl.BlockSpec((_BB, _D), lambda i: (i, 0)),
            pl.BlockSpec((_BB, _N, _D), lambda i: (i, 0, 0)),
            yspec(0), yspec(1), yspec(2), yspec(3),
            gspec(0), gspec(1), gspec(2), gspec(3),
        ],
        out_specs=pl.BlockSpec((1, 1), lambda i: (0, 0)),
        out_shape=jax.ShapeDtypeStruct((1, 1), jnp.float32),
    )(query_emb, items_emb, yg4, yg4, yg4, yg4,
      gates4, gates4, gates4, gates4)
    return out[0, 0] / (_B * 4)


def kernel(query_emb, items_emb, item_text_tokens, item_llm_tokens,
           Wr_text, W1_text, W2_text, Wr_llm, W1_llm, W2_llm):
    xt = item_text_tokens.reshape(_T, _D)
    xl = item_llm_tokens.reshape(_T, _D)
    dest, gates, bexp = _router(xt, xl, Wr_text, Wr_llm)
    dest_flat = dest.reshape(4 * _T)
    xs = _sc_dispatch(xt, xl, dest_flat)
    w1cat = jnp.concatenate([W1_text, W1_llm], axis=0).astype(jnp.bfloat16)
    w2cat = jnp.concatenate([W2_text, W2_llm], axis=0).astype(jnp.bfloat16)
    ys = _ffn(xs, w1cat, w2cat, bexp)
    yg = _sc_gather(ys, dest_flat)
    yg4 = yg.reshape(4, _T, _D)
    gates4 = gates.reshape(4, _T, 1)
    return _loss(query_emb, items_emb, yg4, gates4)


# speedup vs baseline: 1.6151x; 1.4799x over previous
import jax
import jax.numpy as jnp
from jax.experimental import pallas as pl

_B, _N, _D, _DFF, _E = 1024, 5, 256, 512, 8
_TEMP = 0.07
_BB = 256
_TT = _BB * _N


def _top2_gates(logits):
    ii = jax.lax.broadcasted_iota(jnp.int32, logits.shape, 1)
    m1 = jnp.max(logits, axis=1, keepdims=True)
    i1 = jnp.min(jnp.where(logits == m1, ii, _E), axis=1, keepdims=True)
    oh1 = ii == i1
    l2 = jnp.where(oh1, -jnp.inf, logits)
    m2 = jnp.max(l2, axis=1, keepdims=True)
    i2 = jnp.min(jnp.where(l2 == m2, ii, _E), axis=1, keepdims=True)
    oh2 = ii == i2
    e21 = jnp.exp(m2 - m1)
    g1 = 1.0 / (1.0 + e21)
    g2 = e21 * g1
    return jnp.where(oh1, g1, 0.0) + jnp.where(oh2, g2, 0.0)


def _mm(a, b):
    return jax.lax.dot_general(a, b, (((1,), (0,)), ((), ())),
                               preferred_element_type=jnp.float32)


def _moe(x, Wr, W1, W2):
    logits = _mm(x, Wr)
    fg = _top2_gates(logits).astype(jnp.bfloat16)
    xb = x.astype(jnp.bfloat16)
    acc = jnp.zeros((x.shape[0], _D), jnp.bfloat16)
    for e in range(_E):
        h = jax.nn.gelu(_mm(xb, W1[e]).astype(jnp.bfloat16))
        acc = acc + fg[:, e:e + 1] * _mm(h, W2[e]).astype(jnp.bfloat16)
    return acc.astype(jnp.float32)


def _l2n(x):
    n = jnp.sqrt(jnp.sum(x * x, axis=-1, keepdims=True))
    return x / jnp.maximum(n, 1e-12)


def _body(q_ref, items_ref, xt_ref, xl_ref,
          wrt_ref, w1t_ref, w2t_ref, wrl_ref, w1l_ref, w2l_ref, out_ref):
    i = pl.program_id(0)

    @pl.when(i == 0)
    def _():
        out_ref[...] = jnp.zeros((1, 1), jnp.float32)

    xt = xt_ref[...].reshape(_TT, _D)
    xl = xl_ref[...].reshape(_TT, _D)
    yt = _moe(xt, wrt_ref[...], w1t_ref, w2t_ref).reshape(_BB, _N, _D)
    yl = _moe(xl, wrl_ref[...], w1l_ref, w2l_ref).reshape(_BB, _N, _D)

    items = items_ref[...]
    q = q_ref[...]
    pos = jnp.concatenate(
        [q[:, None, :], items[:, 0:1], yl[:, 0:1], yt[:, 0:1]], axis=1)
    neg = jnp.concatenate([items[:, 1:], yl[:, 1:], yt[:, 1:]], axis=1)
    allf = jnp.concatenate([pos, neg], axis=1)
    rn = 1.0 / jnp.maximum(jnp.sqrt(jnp.sum(allf * allf, axis=2)), 1e-12)
    rn_pos = rn[:, :4]

    iota_a = jax.lax.broadcasted_iota(jnp.int32, (_BB, 4), 1)
    exp_pos = jnp.zeros((_BB, 4), jnp.float32)
    exp_neg = jnp.zeros((_BB, 4), jnp.float32)
    for k in range(16):
        s = jnp.sum(pos * allf[:, k:k + 1, :], axis=2)
        s = s * rn_pos * rn[:, k:k + 1]
        ek = jnp.exp(s / _TEMP)
        if k < 4:
            exp_pos = exp_pos + jnp.where(iota_a == k, 0.0, ek)
        else:
            exp_neg = exp_neg + ek
    ratio = exp_pos / (exp_pos + exp_neg + 1e-8)
    ratio = jnp.where(jnp.isnan(ratio), 0.0, ratio)
    out_ref[...] += -jnp.sum(jnp.log(ratio)).reshape(1, 1)


def kernel(query_emb, items_emb, item_text_tokens, item_llm_tokens,
           Wr_text, W1_text, W2_text, Wr_llm, W1_llm, W2_llm):
    grid = _B // _BB
    tok3 = pl.BlockSpec((_BB, _N, _D), lambda i: (i, 0, 0))
    full = lambda shape: pl.BlockSpec(shape, lambda i: (0,) * len(shape))
    out = pl.pallas_call(
        _body,
        grid=(grid,),
        in_specs=[
            pl.BlockSpec((_BB, _D), lambda i: (i, 0)),
            tok3, tok3, tok3,
            full((_D, _E)), full((_E, _D, _DFF)), full((_E, _DFF, _D)),
            full((_D, _E)), full((_E, _D, _DFF)), full((_E, _DFF, _D)),
        ],
        out_specs=pl.BlockSpec((1, 1), lambda i: (0, 0)),
        out_shape=jax.ShapeDtypeStruct((1, 1), jnp.float32),
    )(query_emb, items_emb, item_text_tokens, item_llm_tokens,
      Wr_text, W1_text.astype(jnp.bfloat16), W2_text.astype(jnp.bfloat16),
      Wr_llm, W1_llm.astype(jnp.bfloat16), W2_llm.astype(jnp.bfloat16))
    return out[0, 0] / (_B * 4)


# speedup vs baseline: 1.8826x; 1.1656x over previous
import jax
import jax.numpy as jnp
from jax.experimental import pallas as pl

_B, _N, _D, _DFF, _E = 1024, 5, 256, 512, 8
_TEMP = 0.07
_BB = 128
_TT = _BB * _N


def _top2_gates(logits):
    ii = jax.lax.broadcasted_iota(jnp.int32, logits.shape, 1)
    m1 = jnp.max(logits, axis=1, keepdims=True)
    i1 = jnp.min(jnp.where(logits == m1, ii, _E), axis=1, keepdims=True)
    oh1 = ii == i1
    l2 = jnp.where(oh1, -jnp.inf, logits)
    m2 = jnp.max(l2, axis=1, keepdims=True)
    i2 = jnp.min(jnp.where(l2 == m2, ii, _E), axis=1, keepdims=True)
    oh2 = ii == i2
    e21 = jnp.exp(m2 - m1)
    g1 = 1.0 / (1.0 + e21)
    g2 = e21 * g1
    return jnp.where(oh1, g1, 0.0) + jnp.where(oh2, g2, 0.0)


def _mm(a, b):
    return jax.lax.dot_general(a, b, (((1,), (0,)), ((), ())),
                               preferred_element_type=jnp.float32)


def _moe(x, Wr, W1, W2):
    logits = _mm(x, Wr)
    fg = _top2_gates(logits).astype(jnp.bfloat16)
    xb = x.astype(jnp.bfloat16)
    acc = jnp.zeros((x.shape[0], _D), jnp.bfloat16)
    for e in range(_E):
        h = jax.nn.gelu(_mm(xb, W1[e]).astype(jnp.bfloat16))
        acc = acc + fg[:, e:e + 1] * _mm(h, W2[e]).astype(jnp.bfloat16)
    return acc.astype(jnp.float32)


def _l2n(x):
    n = jnp.sqrt(jnp.sum(x * x, axis=-1, keepdims=True))
    return x / jnp.maximum(n, 1e-12)


def _body(q_ref, items_ref, xt_ref, xl_ref,
          wrt_ref, w1t_ref, w2t_ref, wrl_ref, w1l_ref, w2l_ref, out_ref):
    i = pl.program_id(0)

    @pl.when(i == 0)
    def _():
        out_ref[...] = jnp.zeros((1, 1), jnp.float32)

    xt = xt_ref[...].reshape(_TT, _D)
    xl = xl_ref[...].reshape(_TT, _D)
    yt = _moe(xt, wrt_ref[...], w1t_ref, w2t_ref).reshape(_BB, _N, _D)
    yl = _moe(xl, wrl_ref[...], w1l_ref, w2l_ref).reshape(_BB, _N, _D)

    items = items_ref[...]
    q = q_ref[...]
    pos = jnp.concatenate(
        [q[:, None, :], items[:, 0:1], yl[:, 0:1], yt[:, 0:1]], axis=1)
    neg = jnp.concatenate([items[:, 1:], yl[:, 1:], yt[:, 1:]], axis=1)
    pos = _l2n(pos)
    neg = _l2n(neg)
    allf = jnp.concatenate([pos, neg], axis=1)

    iota_a = jax.lax.broadcasted_iota(jnp.int32, (_BB, 4), 1)
    exp_pos = jnp.zeros((_BB, 4), jnp.float32)
    exp_neg = jnp.zeros((_BB, 4), jnp.float32)
    for k in range(16):
        s = jnp.sum(pos * allf[:, k:k + 1, :], axis=2)
        ek = jnp.exp(s / _TEMP)
        if k < 4:
            exp_pos = exp_pos + jnp.where(iota_a == k, 0.0, ek)
        else:
            exp_neg = exp_neg + ek
    ratio = exp_pos / (exp_pos + exp_neg + 1e-8)
    ratio = jnp.where(jnp.isnan(ratio), 0.0, ratio)
    out_ref[...] += -jnp.sum(jnp.log(ratio)).reshape(1, 1)


def kernel(query_emb, items_emb, item_text_tokens, item_llm_tokens,
           Wr_text, W1_text, W2_text, Wr_llm, W1_llm, W2_llm):
    grid = _B // _BB
    tok3 = pl.BlockSpec((_BB, _N, _D), lambda i: (i, 0, 0))
    full = lambda shape: pl.BlockSpec(shape, lambda i: (0,) * len(shape))
    out = pl.pallas_call(
        _body,
        grid=(grid,),
        in_specs=[
            pl.BlockSpec((_BB, _D), lambda i: (i, 0)),
            tok3, tok3, tok3,
            full((_D, _E)), full((_E, _D, _DFF)), full((_E, _DFF, _D)),
            full((_D, _E)), full((_E, _D, _DFF)), full((_E, _DFF, _D)),
        ],
        out_specs=pl.BlockSpec((1, 1), lambda i: (0, 0)),
        out_shape=jax.ShapeDtypeStruct((1, 1), jnp.float32),
    )(query_emb, items_emb, item_text_tokens, item_llm_tokens,
      Wr_text, W1_text.astype(jnp.bfloat16), W2_text.astype(jnp.bfloat16),
      Wr_llm, W1_llm.astype(jnp.bfloat16), W2_llm.astype(jnp.bfloat16))
    return out[0, 0] / (_B * 4)
